# baseline (speedup 1.0000x reference)
import functools

import jax
import jax.numpy as jnp
from jax import lax
from jax.experimental import pallas as pl
from jax.experimental.pallas import tpu as pltpu
from jax.experimental.pallas import tpu_sc as plsc

NB = 128
C = 384
D = 28 * 28
NC, NS, L = 2, 16, 16
NW = NC * NS
BPW = NB // NW
CHUNK = 64
NCH = C // CHUNK
T = BPW * NCH
NBUF = 2


_mesh = plsc.VectorSubcoreMesh(
    core_axis_name="c", subcore_axis_name="s", num_cores=NC, num_subcores=NS
)


@functools.partial(
    pl.kernel,
    mesh=_mesh,
    out_type=jax.ShapeDtypeStruct((NB * C, D), jnp.float32),
    scratch_types=[
        pltpu.VMEM((C,), jnp.int32),
        pltpu.VMEM((T, CHUNK), jnp.int32),
        pltpu.VMEM((NBUF, CHUNK, D), jnp.float32),
        pltpu.SemaphoreType.DMA,
        pltpu.SemaphoreType.DMA,
    ],
    compiler_params=pltpu.CompilerParams(use_tc_tiling_on_sc=False),
)
def _shuffle(in_hbm, idx_hbm, out_hbm, idx_all, idx_tab, buf, gsem, wsem):
    wid = lax.axis_index("s") * NC + lax.axis_index("c")
    row0 = wid * (BPW * C)

    pltpu.sync_copy(idx_hbm, idx_all)
    for t in range(T):
        base = (wid * BPW + t // NCH) * C
        c0 = (t % NCH) * CHUNK
        for j in range(CHUNK // L):
            v = idx_all[pl.ds(c0 + j * L, L)] + base
            idx_tab[t, pl.ds(j * L, L)] = v

    def gather(t):
        h = pltpu.make_async_copy(
            in_hbm.at[idx_tab.at[t]], buf.at[t % NBUF], gsem
        )
        h.start()
        return h

    def write(t):
        h = pltpu.make_async_copy(
            buf.at[t % NBUF], out_hbm.at[pl.ds(row0 + t * CHUNK, CHUNK)], wsem
        )
        h.start()
        return h

    gh = [None] * T
    wh = [None] * T
    for t in range(min(NBUF, T)):
        gh[t] = gather(t)
    for t in range(T):
        gh[t].wait()
        wh[t] = write(t)
        if t + NBUF < T:
            wh[t].wait()
            gh[t + NBUF] = gather(t + NBUF)
    for t in range(max(0, T - NBUF), T):
        wh[t].wait()


def kernel(input, indices):
    table = input.reshape(NB * C, D)
    out = _shuffle(table, indices.astype(jnp.int32))
    return out.reshape(NB, C, 28, 28)
